# Initial kernel scaffold; baseline (speedup 1.0000x reference)
import jax
import jax.numpy as jnp
from jax.experimental import pallas as pl


def kernel(pred_logits, pred_boxes, pre_boxes, src_valid_trans, patch_area):
    raise NotImplementedError("write your pallas kernel here")


# speedup vs baseline: 16.7074x; 16.7074x over previous
import jax
import jax.numpy as jnp
from jax import lax
from jax.experimental import pallas as pl

N = 5000
NMS_THRE = 0.5
B = 128
NB = 40
NP = NB * B
RWH = 128.0


def _nms_body(boxes_ref, keep_ref):
    b = pl.program_id(0)

    @pl.when(b == 0)
    def _init():
        keep_ref[:, :] = jnp.zeros((NB, B), jnp.float32)

    cur = boxes_ref[pl.ds(b, 1)][0]
    x1r, y1r = cur[0:1, :], cur[1:2, :]
    x2r, y2r = cur[2:3, :], cur[3:4, :]
    area_r = jnp.maximum(x2r - x1r, 0.0) * jnp.maximum(y2r - y1r, 0.0)

    sub_i = lax.broadcasted_iota(jnp.int32, (B, B), 0)
    lane_i = lax.broadcasted_iota(jnp.int32, (B, B), 1)
    eye = (sub_i == lane_i).astype(jnp.float32)

    def to_col(row):
        return jnp.sum(row * eye, axis=1, keepdims=True)

    x1c, y1c = to_col(x1r), to_col(y1r)
    x2c, y2c = to_col(x2r), to_col(y2r)
    area_c = to_col(area_r)

    def iou_tile(ox1, oy1, ox2, oy2, oarea):
        xx1 = jnp.maximum(x1c, ox1)
        yy1 = jnp.maximum(y1c, oy1)
        xx2 = jnp.minimum(x2c, ox2)
        yy2 = jnp.minimum(y2c, oy2)
        inter = jnp.maximum(xx2 - xx1, 0.0) * jnp.maximum(yy2 - yy1, 0.0)
        union = area_c + oarea - inter
        return inter / (union + 1e-9)

    def cross(a, sup):
        oth = boxes_ref[pl.ds(a, 1)][0]
        ox1, oy1 = oth[0:1, :], oth[1:2, :]
        ox2, oy2 = oth[2:3, :], oth[3:4, :]
        oarea = jnp.maximum(ox2 - ox1, 0.0) * jnp.maximum(oy2 - oy1, 0.0)
        iou_x = iou_tile(ox1, oy1, ox2, oy2, oarea)
        keep_o = keep_ref[pl.ds(a, 1), :]
        hit = jnp.where((iou_x > NMS_THRE) & (keep_o > 0.5), 1.0, 0.0)
        s = jnp.max(hit, axis=1, keepdims=True)
        return jnp.maximum(sup, jnp.where(a < b, s, 0.0))

    sup_col = lax.fori_loop(0, NB, cross, jnp.zeros((B, 1), jnp.float32))

    cand_row = jnp.sum((1.0 - sup_col) * eye, axis=0, keepdims=True)

    iou_bb = iou_tile(x1r, y1r, x2r, y2r, area_r)
    iou_gt = jnp.where(iou_bb > NMS_THRE, 1.0, 0.0)
    lane_row = lane_i[0:1, :]

    def intra(i, keep_row):
        row_i = jnp.max(jnp.where(sub_i == i, iou_gt, 0.0), axis=0,
                        keepdims=True)
        k_i = jnp.max(jnp.where(lane_row == i, keep_row, 0.0), axis=1,
                      keepdims=True)
        sup = row_i * jnp.where(lane_row > i, 1.0, 0.0) * k_i
        return keep_row * (1.0 - sup)

    keep_row = lax.fori_loop(0, B, intra, cand_row)
    keep_ref[pl.ds(b, 1), :] = keep_row


def _post_body(pb_ref, pr_ref, aux_ref, aff_ref, out_ref):
    lane = lax.broadcasted_iota(jnp.int32, (1, B), 1)

    def affc(k):
        return jnp.max(jnp.where(lane == k, aff_ref[0:1, :], 0.0), axis=1,
                       keepdims=True)

    a00, a01, a02 = affc(0), affc(1), affc(2)
    a10, a11, a12 = affc(3), affc(4), affc(5)

    def pre2samp_rows(ref):
        r = ref[:, :]
        ctx = r[0:1, :] * RWH
        cty = r[1:2, :] * RWH
        tx = (a00 * ctx + a01 * cty) + a02
        ty = (a10 * ctx + a11 * cty) + a12
        am0 = (r[2:3, :] * RWH) * a00
        am1 = (r[3:4, :] * RWH) * a11
        am2 = (r[4:5, :] * RWH) * a00
        am3 = (r[5:6, :] * RWH) * a11
        return tx, ty, am0, am1, am2, am3

    tx, ty, am0, am1, am2, am3 = pre2samp_rows(pb_ref)
    ptx, pty, pm0, pm1, pm2, pm3 = pre2samp_rows(pr_ref)

    scores = jax.nn.sigmoid(aux_ref[0:1, :])
    keep = aux_ref[1:2, :]

    rows = jnp.concatenate([
        scores,
        tx - am0, ty - am1, tx + am2, ty + am3,
        tx, ty,
        ptx - pm0, pty - pm1, ptx + pm2, pty + pm3,
        jnp.zeros((5, pb_ref.shape[1]), jnp.float32),
    ], axis=0) * keep
    out_ref[:, :] = rows


def kernel(pred_logits, pred_boxes, pre_boxes, src_valid_trans, patch_area):
    logit1 = pred_logits[0, :, 1]
    scores_nms = jax.nn.sigmoid(logit1)
    order = jnp.argsort(-scores_nms)

    pb0 = pred_boxes[0]
    nnpp = jnp.array([-1.0, -1.0, 1.0, 1.0], jnp.float32)
    rect_copy = pb0[:, 2:6] * nnpp + jnp.tile(pb0[:, 0:2], (1, 2))
    rect_sorted = rect_copy[order]
    rect_pad = jnp.zeros((NP, 4), jnp.float32).at[:N].set(rect_sorted)
    bs = rect_pad.T.reshape(4, NB, B).transpose(1, 0, 2)
    bs = jnp.concatenate([bs, jnp.zeros((NB, 4, B), jnp.float32)], axis=1)

    keep_blocks = pl.pallas_call(
        _nms_body,
        grid=(NB,),
        in_specs=[pl.BlockSpec((NB, 8, B), lambda i: (0, 0, 0))],
        out_specs=pl.BlockSpec((NB, B), lambda i: (0, 0)),
        out_shape=jax.ShapeDtypeStruct((NB, B), jnp.float32),
    )(bs)

    keep_sorted = keep_blocks.reshape(NP)[:N]
    keep = jnp.zeros((N,), jnp.float32).at[order].set(keep_sorted)

    def chan_major(x):
        t = jnp.zeros((8, NP), jnp.float32)
        return t.at[:6, :N].set(x.T)

    pbT = chan_major(pb0)
    prT = chan_major(pre_boxes[0])
    aux = jnp.zeros((8, NP), jnp.float32).at[0, :N].set(logit1).at[1, :N].set(keep)
    aff = jnp.zeros((8, B), jnp.float32).at[0, :6].set(
        src_valid_trans[0, 0].reshape(6))

    res = pl.pallas_call(
        _post_body,
        in_specs=[pl.BlockSpec((8, NP), lambda: (0, 0)),
                  pl.BlockSpec((8, NP), lambda: (0, 0)),
                  pl.BlockSpec((8, NP), lambda: (0, 0)),
                  pl.BlockSpec((8, B), lambda: (0, 0))],
        out_specs=pl.BlockSpec((16, NP), lambda: (0, 0)),
        out_shape=jax.ShapeDtypeStruct((16, NP), jnp.float32),
    )(pbT, prT, aux, aff)

    out = res[:11, :N].T[None]
    return out, patch_area


# speedup vs baseline: 20.0183x; 1.1982x over previous
import jax
import jax.numpy as jnp
from jax import lax
from jax.experimental import pallas as pl
from jax.experimental.pallas import tpu as pltpu

N = 5000
NMS_THRE = 0.5
B = 128
NB = 40
NP = NB * B
RWH = 128.0


def _nms_body(boxes_ref, keep_ref, iou_scr):
    b = pl.program_id(0)

    @pl.when(b == 0)
    def _init():
        keep_ref[:, :] = jnp.zeros((NB, B), jnp.float32)

    cur = boxes_ref[pl.ds(b, 1)][0]
    x1r, y1r = cur[0:1, :], cur[1:2, :]
    x2r, y2r = cur[2:3, :], cur[3:4, :]
    area_r = jnp.maximum(x2r - x1r, 0.0) * jnp.maximum(y2r - y1r, 0.0)

    sub_i = lax.broadcasted_iota(jnp.int32, (B, B), 0)
    lane_i = lax.broadcasted_iota(jnp.int32, (B, B), 1)
    eye = (sub_i == lane_i).astype(jnp.float32)

    def to_col(row):
        return jnp.sum(row * eye, axis=1, keepdims=True)

    x1c, y1c = to_col(x1r), to_col(y1r)
    x2c, y2c = to_col(x2r), to_col(y2r)
    area_c = to_col(area_r)

    def iou_tile(ox1, oy1, ox2, oy2, oarea):
        xx1 = jnp.maximum(x1c, ox1)
        yy1 = jnp.maximum(y1c, oy1)
        xx2 = jnp.minimum(x2c, ox2)
        yy2 = jnp.minimum(y2c, oy2)
        inter = jnp.maximum(xx2 - xx1, 0.0) * jnp.maximum(yy2 - yy1, 0.0)
        union = area_c + oarea - inter
        return inter / (union + 1e-9)

    def cross(a, sup):
        oth = boxes_ref[pl.ds(a, 1)][0]
        ox1, oy1 = oth[0:1, :], oth[1:2, :]
        ox2, oy2 = oth[2:3, :], oth[3:4, :]
        oarea = jnp.maximum(ox2 - ox1, 0.0) * jnp.maximum(oy2 - oy1, 0.0)
        iou_x = iou_tile(ox1, oy1, ox2, oy2, oarea)
        keep_o = keep_ref[pl.ds(a, 1), :]
        hit = jnp.where((iou_x > NMS_THRE) & (keep_o > 0.5), 1.0, 0.0)
        s = jnp.max(hit, axis=1, keepdims=True)
        return jnp.maximum(sup, s)

    sup_col = lax.fori_loop(0, b, cross, jnp.zeros((B, 1), jnp.float32))

    cand_row = jnp.sum((1.0 - sup_col) * eye, axis=0, keepdims=True)

    iou_bb = iou_tile(x1r, y1r, x2r, y2r, area_r)
    iou_scr[:, :] = jnp.where(iou_bb > NMS_THRE, 1.0, 0.0)
    lane_row = lane_i[0:1, :]

    def intra(i, keep_row):
        row_i = iou_scr[pl.ds(i, 1), :]
        k_i = jnp.max(jnp.where(lane_row == i, keep_row, 0.0), axis=1,
                      keepdims=True)
        sup = row_i * jnp.where(lane_row > i, 1.0, 0.0) * k_i
        return keep_row * (1.0 - sup)

    keep_row = lax.fori_loop(0, B, intra, cand_row)
    keep_ref[pl.ds(b, 1), :] = keep_row


def _post_body(pb_ref, pr_ref, aux_ref, aff_ref, out_ref):
    lane = lax.broadcasted_iota(jnp.int32, (1, B), 1)

    def affc(k):
        return jnp.max(jnp.where(lane == k, aff_ref[0:1, :], 0.0), axis=1,
                       keepdims=True)

    a00, a01, a02 = affc(0), affc(1), affc(2)
    a10, a11, a12 = affc(3), affc(4), affc(5)

    def pre2samp_rows(ref):
        r = ref[:, :]
        ctx = r[0:1, :] * RWH
        cty = r[1:2, :] * RWH
        tx = (a00 * ctx + a01 * cty) + a02
        ty = (a10 * ctx + a11 * cty) + a12
        am0 = (r[2:3, :] * RWH) * a00
        am1 = (r[3:4, :] * RWH) * a11
        am2 = (r[4:5, :] * RWH) * a00
        am3 = (r[5:6, :] * RWH) * a11
        return tx, ty, am0, am1, am2, am3

    tx, ty, am0, am1, am2, am3 = pre2samp_rows(pb_ref)
    ptx, pty, pm0, pm1, pm2, pm3 = pre2samp_rows(pr_ref)

    scores = jax.nn.sigmoid(aux_ref[0:1, :])
    keep = aux_ref[1:2, :]

    rows = jnp.concatenate([
        scores,
        tx - am0, ty - am1, tx + am2, ty + am3,
        tx, ty,
        ptx - pm0, pty - pm1, ptx + pm2, pty + pm3,
        jnp.zeros((5, pb_ref.shape[1]), jnp.float32),
    ], axis=0) * keep
    out_ref[:, :] = rows


def kernel(pred_logits, pred_boxes, pre_boxes, src_valid_trans, patch_area):
    logit1 = pred_logits[0, :, 1]
    scores_nms = jax.nn.sigmoid(logit1)
    order = jnp.argsort(-scores_nms)

    pb0 = pred_boxes[0]
    nnpp = jnp.array([-1.0, -1.0, 1.0, 1.0], jnp.float32)
    rect_copy = pb0[:, 2:6] * nnpp + jnp.tile(pb0[:, 0:2], (1, 2))
    rect_sorted = rect_copy[order]
    rect_pad = jnp.zeros((NP, 4), jnp.float32).at[:N].set(rect_sorted)
    bs = rect_pad.T.reshape(4, NB, B).transpose(1, 0, 2)
    bs = jnp.concatenate([bs, jnp.zeros((NB, 4, B), jnp.float32)], axis=1)

    keep_blocks = pl.pallas_call(
        _nms_body,
        grid=(NB,),
        in_specs=[pl.BlockSpec((NB, 8, B), lambda i: (0, 0, 0))],
        out_specs=pl.BlockSpec((NB, B), lambda i: (0, 0)),
        out_shape=jax.ShapeDtypeStruct((NB, B), jnp.float32),
        scratch_shapes=[pltpu.VMEM((B, B), jnp.float32)],
    )(bs)

    keep_sorted = keep_blocks.reshape(NP)[:N]
    keep = jnp.zeros((N,), jnp.float32).at[order].set(keep_sorted)

    def chan_major(x):
        t = jnp.zeros((8, NP), jnp.float32)
        return t.at[:6, :N].set(x.T)

    pbT = chan_major(pb0)
    prT = chan_major(pre_boxes[0])
    aux = jnp.zeros((8, NP), jnp.float32).at[0, :N].set(logit1).at[1, :N].set(keep)
    aff = jnp.zeros((8, B), jnp.float32).at[0, :6].set(
        src_valid_trans[0, 0].reshape(6))

    res = pl.pallas_call(
        _post_body,
        in_specs=[pl.BlockSpec((8, NP), lambda: (0, 0)),
                  pl.BlockSpec((8, NP), lambda: (0, 0)),
                  pl.BlockSpec((8, NP), lambda: (0, 0)),
                  pl.BlockSpec((8, B), lambda: (0, 0))],
        out_specs=pl.BlockSpec((16, NP), lambda: (0, 0)),
        out_shape=jax.ShapeDtypeStruct((16, NP), jnp.float32),
    )(pbT, prT, aux, aff)

    out = res[:11, :N].T[None]
    return out, patch_area


# speedup vs baseline: 20.2398x; 1.0111x over previous
import jax
import jax.numpy as jnp
from jax import lax
from jax.experimental import pallas as pl
from jax.experimental.pallas import tpu as pltpu

N = 5000
NMS_THRE = 0.5
B = 128
NB = 40
NP = NB * B
RWH = 128.0


def _nms_body(boxes_ref, keep_ref, iou_scr):
    b = pl.program_id(0)

    @pl.when(b == 0)
    def _init():
        keep_ref[:, :] = jnp.zeros((NB, B), jnp.float32)

    cur = boxes_ref[pl.ds(b, 1)][0]
    x1r, y1r = cur[0:1, :], cur[1:2, :]
    x2r, y2r = cur[2:3, :], cur[3:4, :]
    area_r = jnp.maximum(x2r - x1r, 0.0) * jnp.maximum(y2r - y1r, 0.0)

    sub_i = lax.broadcasted_iota(jnp.int32, (B, B), 0)
    lane_i = lax.broadcasted_iota(jnp.int32, (B, B), 1)
    eye = (sub_i == lane_i).astype(jnp.float32)

    def to_col(row):
        return jnp.sum(row * eye, axis=1, keepdims=True)

    x1c, y1c = to_col(x1r), to_col(y1r)
    x2c, y2c = to_col(x2r), to_col(y2r)
    area_c = to_col(area_r)

    def iou_hit(ox1, oy1, ox2, oy2, oarea):
        xx1 = jnp.maximum(x1c, ox1)
        yy1 = jnp.maximum(y1c, oy1)
        xx2 = jnp.minimum(x2c, ox2)
        yy2 = jnp.minimum(y2c, oy2)
        inter = jnp.maximum(xx2 - xx1, 0.0) * jnp.maximum(yy2 - yy1, 0.0)
        union = area_c + oarea - inter
        return inter > NMS_THRE * (union + 1e-9)

    def cross(a, sup):
        oth = boxes_ref[pl.ds(a, 1)][0]
        ox1, oy1 = oth[0:1, :], oth[1:2, :]
        ox2, oy2 = oth[2:3, :], oth[3:4, :]
        oarea = jnp.maximum(ox2 - ox1, 0.0) * jnp.maximum(oy2 - oy1, 0.0)
        iou_x = iou_hit(ox1, oy1, ox2, oy2, oarea)
        keep_o = keep_ref[pl.ds(a, 1), :]
        hit = jnp.where(iou_x & (keep_o > 0.5), 1.0, 0.0)
        s = jnp.max(hit, axis=1, keepdims=True)
        return jnp.maximum(sup, s)

    sup_col = lax.fori_loop(0, b, cross, jnp.zeros((B, 1), jnp.float32))

    cand_row = jnp.sum((1.0 - sup_col) * eye, axis=0, keepdims=True)

    iou_bb = iou_hit(x1r, y1r, x2r, y2r, area_r)
    iou_scr[:, :] = jnp.where(iou_bb & (sub_i < lane_i), 1.0, 0.0)
    lane_row = lane_i[0:1, :]

    def intra(i, keep_row):
        row_i = iou_scr[pl.ds(i, 1), :]
        k_i = jnp.max(jnp.where(lane_row == i, keep_row, 0.0), axis=1,
                      keepdims=True)
        return keep_row * (1.0 - row_i * k_i)

    keep_row = lax.fori_loop(0, B, intra, cand_row)
    keep_ref[pl.ds(b, 1), :] = keep_row


def _post_body(pb_ref, pr_ref, aux_ref, aff_ref, out_ref):
    lane = lax.broadcasted_iota(jnp.int32, (1, B), 1)

    def affc(k):
        return jnp.max(jnp.where(lane == k, aff_ref[0:1, :], 0.0), axis=1,
                       keepdims=True)

    a00, a01, a02 = affc(0), affc(1), affc(2)
    a10, a11, a12 = affc(3), affc(4), affc(5)

    def pre2samp_rows(ref):
        r = ref[:, :]
        ctx = r[0:1, :] * RWH
        cty = r[1:2, :] * RWH
        tx = (a00 * ctx + a01 * cty) + a02
        ty = (a10 * ctx + a11 * cty) + a12
        am0 = (r[2:3, :] * RWH) * a00
        am1 = (r[3:4, :] * RWH) * a11
        am2 = (r[4:5, :] * RWH) * a00
        am3 = (r[5:6, :] * RWH) * a11
        return tx, ty, am0, am1, am2, am3

    tx, ty, am0, am1, am2, am3 = pre2samp_rows(pb_ref)
    ptx, pty, pm0, pm1, pm2, pm3 = pre2samp_rows(pr_ref)

    scores = jax.nn.sigmoid(aux_ref[0:1, :])
    keep = aux_ref[1:2, :]

    rows = jnp.concatenate([
        scores,
        tx - am0, ty - am1, tx + am2, ty + am3,
        tx, ty,
        ptx - pm0, pty - pm1, ptx + pm2, pty + pm3,
        jnp.zeros((5, pb_ref.shape[1]), jnp.float32),
    ], axis=0) * keep
    out_ref[:, :] = rows


def kernel(pred_logits, pred_boxes, pre_boxes, src_valid_trans, patch_area):
    logit1 = pred_logits[0, :, 1]
    scores_nms = jax.nn.sigmoid(logit1)
    order = jnp.argsort(-scores_nms)

    pb0 = pred_boxes[0]
    nnpp = jnp.array([-1.0, -1.0, 1.0, 1.0], jnp.float32)
    rect_copy = pb0[:, 2:6] * nnpp + jnp.tile(pb0[:, 0:2], (1, 2))
    rect_sorted = rect_copy[order]
    rect_pad = jnp.zeros((NP, 4), jnp.float32).at[:N].set(rect_sorted)
    bs = rect_pad.T.reshape(4, NB, B).transpose(1, 0, 2)
    bs = jnp.concatenate([bs, jnp.zeros((NB, 4, B), jnp.float32)], axis=1)

    keep_blocks = pl.pallas_call(
        _nms_body,
        grid=(NB,),
        in_specs=[pl.BlockSpec((NB, 8, B), lambda i: (0, 0, 0))],
        out_specs=pl.BlockSpec((NB, B), lambda i: (0, 0)),
        out_shape=jax.ShapeDtypeStruct((NB, B), jnp.float32),
        scratch_shapes=[pltpu.VMEM((B, B), jnp.float32)],
    )(bs)

    keep_sorted = keep_blocks.reshape(NP)[:N]
    keep = jnp.zeros((N,), jnp.float32).at[order].set(keep_sorted)

    def chan_major(x):
        t = jnp.zeros((8, NP), jnp.float32)
        return t.at[:6, :N].set(x.T)

    pbT = chan_major(pb0)
    prT = chan_major(pre_boxes[0])
    aux = jnp.zeros((8, NP), jnp.float32).at[0, :N].set(logit1).at[1, :N].set(keep)
    aff = jnp.zeros((8, B), jnp.float32).at[0, :6].set(
        src_valid_trans[0, 0].reshape(6))

    res = pl.pallas_call(
        _post_body,
        in_specs=[pl.BlockSpec((8, NP), lambda: (0, 0)),
                  pl.BlockSpec((8, NP), lambda: (0, 0)),
                  pl.BlockSpec((8, NP), lambda: (0, 0)),
                  pl.BlockSpec((8, B), lambda: (0, 0))],
        out_specs=pl.BlockSpec((16, NP), lambda: (0, 0)),
        out_shape=jax.ShapeDtypeStruct((16, NP), jnp.float32),
    )(pbT, prT, aux, aff)

    out = res[:11, :N].T[None]
    return out, patch_area


# speedup vs baseline: 64.2203x; 3.1730x over previous
import jax
import jax.numpy as jnp
from jax import lax
from jax.experimental import pallas as pl
from jax.experimental.pallas import tpu as pltpu

N = 5000
NMS_THRE = 0.5
B = 128
NB = 40
NP = NB * B
RWH = 128.0


def _nms_body(boxes_ref, keep_ref):
    b = pl.program_id(0)

    @pl.when(b == 0)
    def _init():
        keep_ref[:, :] = jnp.zeros((NB, B), jnp.float32)

    cur = boxes_ref[pl.ds(b, 1)][0]
    x1r, y1r = cur[0:1, :], cur[1:2, :]
    x2r, y2r = cur[2:3, :], cur[3:4, :]
    area_r = jnp.maximum(x2r - x1r, 0.0) * jnp.maximum(y2r - y1r, 0.0)

    sub_i = lax.broadcasted_iota(jnp.int32, (B, B), 0)
    lane_i = lax.broadcasted_iota(jnp.int32, (B, B), 1)
    eye = (sub_i == lane_i).astype(jnp.float32)

    def to_col(row):
        return jnp.sum(row * eye, axis=1, keepdims=True)

    x1c, y1c = to_col(x1r), to_col(y1r)
    x2c, y2c = to_col(x2r), to_col(y2r)
    area_c = to_col(area_r)

    def iou_hit(ox1, oy1, ox2, oy2, oarea):
        xx1 = jnp.maximum(x1c, ox1)
        yy1 = jnp.maximum(y1c, oy1)
        xx2 = jnp.minimum(x2c, ox2)
        yy2 = jnp.minimum(y2c, oy2)
        inter = jnp.maximum(xx2 - xx1, 0.0) * jnp.maximum(yy2 - yy1, 0.0)
        union = area_c + oarea - inter
        return inter > NMS_THRE * (union + 1e-9)

    def cross(a, sup):
        oth = boxes_ref[pl.ds(a, 1)][0]
        ox1, oy1 = oth[0:1, :], oth[1:2, :]
        ox2, oy2 = oth[2:3, :], oth[3:4, :]
        oarea = jnp.maximum(ox2 - ox1, 0.0) * jnp.maximum(oy2 - oy1, 0.0)
        iou_x = iou_hit(ox1, oy1, ox2, oy2, oarea)
        keep_o = keep_ref[pl.ds(a, 1), :]
        hit = jnp.where(iou_x & (keep_o > 0.5), 1.0, 0.0)
        s = jnp.max(hit, axis=1, keepdims=True)
        return jnp.maximum(sup, s)

    sup_col = lax.fori_loop(0, b, cross, jnp.zeros((B, 1), jnp.float32))

    cand_row = jnp.sum((1.0 - sup_col) * eye, axis=0, keepdims=True)

    iou_bb = iou_hit(x1r, y1r, x2r, y2r, area_r)
    u_mat = jnp.where(iou_bb & (sub_i < lane_i), 1.0, 0.0)

    def fp_cond(state):
        t, changed, _ = state
        return jnp.logical_and(changed > 0.5, t < B)

    def fp_body(state):
        t, _, k_row = state
        k_col = jnp.sum(k_row * eye, axis=1, keepdims=True)
        sup = jnp.max(u_mat * k_col, axis=0, keepdims=True)
        k_new = cand_row * (1.0 - sup)
        changed = jnp.max(jnp.abs(k_new - k_row))
        return t + 1, changed, k_new

    _, _, keep_row = lax.while_loop(
        fp_cond, fp_body, (jnp.int32(0), jnp.float32(1.0), cand_row))
    keep_ref[pl.ds(b, 1), :] = keep_row


def _post_body(pb_ref, pr_ref, aux_ref, aff_ref, out_ref):
    lane = lax.broadcasted_iota(jnp.int32, (1, B), 1)

    def affc(k):
        return jnp.max(jnp.where(lane == k, aff_ref[0:1, :], 0.0), axis=1,
                       keepdims=True)

    a00, a01, a02 = affc(0), affc(1), affc(2)
    a10, a11, a12 = affc(3), affc(4), affc(5)

    def pre2samp_rows(ref):
        r = ref[:, :]
        ctx = r[0:1, :] * RWH
        cty = r[1:2, :] * RWH
        tx = (a00 * ctx + a01 * cty) + a02
        ty = (a10 * ctx + a11 * cty) + a12
        am0 = (r[2:3, :] * RWH) * a00
        am1 = (r[3:4, :] * RWH) * a11
        am2 = (r[4:5, :] * RWH) * a00
        am3 = (r[5:6, :] * RWH) * a11
        return tx, ty, am0, am1, am2, am3

    tx, ty, am0, am1, am2, am3 = pre2samp_rows(pb_ref)
    ptx, pty, pm0, pm1, pm2, pm3 = pre2samp_rows(pr_ref)

    scores = jax.nn.sigmoid(aux_ref[0:1, :])
    keep = aux_ref[1:2, :]

    rows = jnp.concatenate([
        scores,
        tx - am0, ty - am1, tx + am2, ty + am3,
        tx, ty,
        ptx - pm0, pty - pm1, ptx + pm2, pty + pm3,
        jnp.zeros((5, pb_ref.shape[1]), jnp.float32),
    ], axis=0) * keep
    out_ref[:, :] = rows


def kernel(pred_logits, pred_boxes, pre_boxes, src_valid_trans, patch_area):
    logit1 = pred_logits[0, :, 1]
    scores_nms = jax.nn.sigmoid(logit1)
    order = jnp.argsort(-scores_nms)

    pb0 = pred_boxes[0]
    nnpp = jnp.array([-1.0, -1.0, 1.0, 1.0], jnp.float32)
    rect_copy = pb0[:, 2:6] * nnpp + jnp.tile(pb0[:, 0:2], (1, 2))
    rect_sorted = rect_copy[order]
    rect_pad = jnp.zeros((NP, 4), jnp.float32).at[:N].set(rect_sorted)
    bs = rect_pad.T.reshape(4, NB, B).transpose(1, 0, 2)
    bs = jnp.concatenate([bs, jnp.zeros((NB, 4, B), jnp.float32)], axis=1)

    keep_blocks = pl.pallas_call(
        _nms_body,
        grid=(NB,),
        in_specs=[pl.BlockSpec((NB, 8, B), lambda i: (0, 0, 0))],
        out_specs=pl.BlockSpec((NB, B), lambda i: (0, 0)),
        out_shape=jax.ShapeDtypeStruct((NB, B), jnp.float32),
    )(bs)

    keep_sorted = keep_blocks.reshape(NP)[:N]
    keep = jnp.zeros((N,), jnp.float32).at[order].set(keep_sorted)

    def chan_major(x):
        t = jnp.zeros((8, NP), jnp.float32)
        return t.at[:6, :N].set(x.T)

    pbT = chan_major(pb0)
    prT = chan_major(pre_boxes[0])
    aux = jnp.zeros((8, NP), jnp.float32).at[0, :N].set(logit1).at[1, :N].set(keep)
    aff = jnp.zeros((8, B), jnp.float32).at[0, :6].set(
        src_valid_trans[0, 0].reshape(6))

    res = pl.pallas_call(
        _post_body,
        in_specs=[pl.BlockSpec((8, NP), lambda: (0, 0)),
                  pl.BlockSpec((8, NP), lambda: (0, 0)),
                  pl.BlockSpec((8, NP), lambda: (0, 0)),
                  pl.BlockSpec((8, B), lambda: (0, 0))],
        out_specs=pl.BlockSpec((16, NP), lambda: (0, 0)),
        out_shape=jax.ShapeDtypeStruct((16, NP), jnp.float32),
    )(pbT, prT, aux, aff)

    out = res[:11, :N].T[None]
    return out, patch_area


# speedup vs baseline: 102.5067x; 1.5962x over previous
import jax
import jax.numpy as jnp
from jax import lax
from jax.experimental import pallas as pl
from jax.experimental.pallas import tpu as pltpu

N = 5000
NMS_THRE = 0.5
B = 512
NB = 10
NP = NB * B
RWH = 128.0


def _nms_body(boxes_ref, keep_ref):
    b = pl.program_id(0)

    @pl.when(b == 0)
    def _init():
        keep_ref[:, :] = jnp.zeros((NB, B), jnp.float32)

    cur = boxes_ref[pl.ds(b, 1)][0]
    x1r, y1r = cur[0:1, :], cur[1:2, :]
    x2r, y2r = cur[2:3, :], cur[3:4, :]
    area_r = jnp.maximum(x2r - x1r, 0.0) * jnp.maximum(y2r - y1r, 0.0)

    sub_i = lax.broadcasted_iota(jnp.int32, (B, B), 0)
    lane_i = lax.broadcasted_iota(jnp.int32, (B, B), 1)
    eye = (sub_i == lane_i).astype(jnp.float32)

    def to_col(row):
        return jnp.sum(row * eye, axis=1, keepdims=True)

    x1c, y1c = to_col(x1r), to_col(y1r)
    x2c, y2c = to_col(x2r), to_col(y2r)
    area_c = to_col(area_r)

    def iou_hit(ox1, oy1, ox2, oy2, oarea):
        xx1 = jnp.maximum(x1c, ox1)
        yy1 = jnp.maximum(y1c, oy1)
        xx2 = jnp.minimum(x2c, ox2)
        yy2 = jnp.minimum(y2c, oy2)
        inter = jnp.maximum(xx2 - xx1, 0.0) * jnp.maximum(yy2 - yy1, 0.0)
        union = area_c + oarea - inter
        return inter > NMS_THRE * (union + 1e-9)

    def cross(a, sup):
        oth = boxes_ref[pl.ds(a, 1)][0]
        ox1, oy1 = oth[0:1, :], oth[1:2, :]
        ox2, oy2 = oth[2:3, :], oth[3:4, :]
        oarea = jnp.maximum(ox2 - ox1, 0.0) * jnp.maximum(oy2 - oy1, 0.0)
        iou_x = iou_hit(ox1, oy1, ox2, oy2, oarea)
        keep_o = keep_ref[pl.ds(a, 1), :]
        hit = jnp.where(iou_x & (keep_o > 0.5), 1.0, 0.0)
        s = jnp.max(hit, axis=1, keepdims=True)
        return jnp.maximum(sup, s)

    sup_col = lax.fori_loop(0, b, cross, jnp.zeros((B, 1), jnp.float32))

    cand_row = jnp.sum((1.0 - sup_col) * eye, axis=0, keepdims=True)

    iou_bb = iou_hit(x1r, y1r, x2r, y2r, area_r)
    u_mat = jnp.where(iou_bb & (sub_i < lane_i), 1.0, 0.0)

    def fp_cond(state):
        t, changed, _ = state
        return jnp.logical_and(changed > 0.5, t < B)

    def fp_body(state):
        t, _, k_row = state
        k_col = jnp.sum(k_row * eye, axis=1, keepdims=True)
        sup = jnp.max(u_mat * k_col, axis=0, keepdims=True)
        k_new = cand_row * (1.0 - sup)
        changed = jnp.max(jnp.abs(k_new - k_row))
        return t + 1, changed, k_new

    _, _, keep_row = lax.while_loop(
        fp_cond, fp_body, (jnp.int32(0), jnp.float32(1.0), cand_row))
    keep_ref[pl.ds(b, 1), :] = keep_row


def _post_body(pb_ref, pr_ref, aux_ref, aff_ref, out_ref):
    lane = lax.broadcasted_iota(jnp.int32, (1, 128), 1)

    def affc(k):
        return jnp.max(jnp.where(lane == k, aff_ref[0:1, :], 0.0), axis=1,
                       keepdims=True)

    a00, a01, a02 = affc(0), affc(1), affc(2)
    a10, a11, a12 = affc(3), affc(4), affc(5)

    def pre2samp_rows(ref):
        r = ref[:, :]
        ctx = r[0:1, :] * RWH
        cty = r[1:2, :] * RWH
        tx = (a00 * ctx + a01 * cty) + a02
        ty = (a10 * ctx + a11 * cty) + a12
        am0 = (r[2:3, :] * RWH) * a00
        am1 = (r[3:4, :] * RWH) * a11
        am2 = (r[4:5, :] * RWH) * a00
        am3 = (r[5:6, :] * RWH) * a11
        return tx, ty, am0, am1, am2, am3

    tx, ty, am0, am1, am2, am3 = pre2samp_rows(pb_ref)
    ptx, pty, pm0, pm1, pm2, pm3 = pre2samp_rows(pr_ref)

    scores = jax.nn.sigmoid(aux_ref[0:1, :])
    keep = aux_ref[1:2, :]

    rows = jnp.concatenate([
        scores,
        tx - am0, ty - am1, tx + am2, ty + am3,
        tx, ty,
        ptx - pm0, pty - pm1, ptx + pm2, pty + pm3,
        jnp.zeros((5, pb_ref.shape[1]), jnp.float32),
    ], axis=0) * keep
    out_ref[:, :] = rows


def kernel(pred_logits, pred_boxes, pre_boxes, src_valid_trans, patch_area):
    logit1 = pred_logits[0, :, 1]
    scores_nms = jax.nn.sigmoid(logit1)
    order = jnp.argsort(-scores_nms)

    pb0 = pred_boxes[0]
    nnpp = jnp.array([-1.0, -1.0, 1.0, 1.0], jnp.float32)
    rect_copy = pb0[:, 2:6] * nnpp + jnp.tile(pb0[:, 0:2], (1, 2))
    rect_sorted = rect_copy[order]
    rect_pad = jnp.zeros((NP, 4), jnp.float32).at[:N].set(rect_sorted)
    bs = rect_pad.T.reshape(4, NB, B).transpose(1, 0, 2)
    bs = jnp.concatenate([bs, jnp.zeros((NB, 4, B), jnp.float32)], axis=1)

    keep_blocks = pl.pallas_call(
        _nms_body,
        grid=(NB,),
        in_specs=[pl.BlockSpec((NB, 8, B), lambda i: (0, 0, 0))],
        out_specs=pl.BlockSpec((NB, B), lambda i: (0, 0)),
        out_shape=jax.ShapeDtypeStruct((NB, B), jnp.float32),
    )(bs)

    keep_sorted = keep_blocks.reshape(NP)[:N]
    keep = jnp.zeros((N,), jnp.float32).at[order].set(keep_sorted)

    def chan_major(x):
        t = jnp.zeros((8, NP), jnp.float32)
        return t.at[:6, :N].set(x.T)

    pbT = chan_major(pb0)
    prT = chan_major(pre_boxes[0])
    aux = jnp.zeros((8, NP), jnp.float32).at[0, :N].set(logit1).at[1, :N].set(keep)
    aff = jnp.zeros((8, 128), jnp.float32).at[0, :6].set(
        src_valid_trans[0, 0].reshape(6))

    res = pl.pallas_call(
        _post_body,
        in_specs=[pl.BlockSpec((8, NP), lambda: (0, 0)),
                  pl.BlockSpec((8, NP), lambda: (0, 0)),
                  pl.BlockSpec((8, NP), lambda: (0, 0)),
                  pl.BlockSpec((8, 128), lambda: (0, 0))],
        out_specs=pl.BlockSpec((16, NP), lambda: (0, 0)),
        out_shape=jax.ShapeDtypeStruct((16, NP), jnp.float32),
    )(pbT, prT, aux, aff)

    out = res[:11, :N].T[None]
    return out, patch_area


# speedup vs baseline: 102.6472x; 1.0014x over previous
import jax
import jax.numpy as jnp
from jax import lax
from jax.experimental import pallas as pl
from jax.experimental.pallas import tpu as pltpu

N = 5000
NMS_THRE = 0.5
B = 1024
NB = 5
NP = NB * B
RWH = 128.0


def _nms_body(boxes_ref, keep_ref):
    b = pl.program_id(0)

    @pl.when(b == 0)
    def _init():
        keep_ref[:, :] = jnp.zeros((NB, B), jnp.float32)

    cur = boxes_ref[pl.ds(b, 1)][0]
    x1r, y1r = cur[0:1, :], cur[1:2, :]
    x2r, y2r = cur[2:3, :], cur[3:4, :]
    area_r = jnp.maximum(x2r - x1r, 0.0) * jnp.maximum(y2r - y1r, 0.0)

    sub_i = lax.broadcasted_iota(jnp.int32, (B, B), 0)
    lane_i = lax.broadcasted_iota(jnp.int32, (B, B), 1)
    eye = (sub_i == lane_i).astype(jnp.float32)

    def to_col(row):
        return jnp.sum(row * eye, axis=1, keepdims=True)

    x1c, y1c = to_col(x1r), to_col(y1r)
    x2c, y2c = to_col(x2r), to_col(y2r)
    area_c = to_col(area_r)

    def iou_hit(ox1, oy1, ox2, oy2, oarea):
        xx1 = jnp.maximum(x1c, ox1)
        yy1 = jnp.maximum(y1c, oy1)
        xx2 = jnp.minimum(x2c, ox2)
        yy2 = jnp.minimum(y2c, oy2)
        inter = jnp.maximum(xx2 - xx1, 0.0) * jnp.maximum(yy2 - yy1, 0.0)
        union = area_c + oarea - inter
        return inter > NMS_THRE * (union + 1e-9)

    def cross(a, sup):
        oth = boxes_ref[pl.ds(a, 1)][0]
        ox1, oy1 = oth[0:1, :], oth[1:2, :]
        ox2, oy2 = oth[2:3, :], oth[3:4, :]
        oarea = jnp.maximum(ox2 - ox1, 0.0) * jnp.maximum(oy2 - oy1, 0.0)
        iou_x = iou_hit(ox1, oy1, ox2, oy2, oarea)
        keep_o = keep_ref[pl.ds(a, 1), :]
        hit = jnp.where(iou_x & (keep_o > 0.5), 1.0, 0.0)
        s = jnp.max(hit, axis=1, keepdims=True)
        return jnp.maximum(sup, s)

    sup_col = lax.fori_loop(0, b, cross, jnp.zeros((B, 1), jnp.float32))

    cand_row = jnp.sum((1.0 - sup_col) * eye, axis=0, keepdims=True)

    iou_bb = iou_hit(x1r, y1r, x2r, y2r, area_r)
    u_mat = jnp.where(iou_bb & (sub_i < lane_i), 1.0, 0.0)

    def fp_cond(state):
        t, changed, _ = state
        return jnp.logical_and(changed > 0.5, t < B)

    def fp_body(state):
        t, _, k_row = state
        k_col = jnp.sum(k_row * eye, axis=1, keepdims=True)
        sup = jnp.max(u_mat * k_col, axis=0, keepdims=True)
        k_new = cand_row * (1.0 - sup)
        changed = jnp.max(jnp.abs(k_new - k_row))
        return t + 1, changed, k_new

    _, _, keep_row = lax.while_loop(
        fp_cond, fp_body, (jnp.int32(0), jnp.float32(1.0), cand_row))
    keep_ref[pl.ds(b, 1), :] = keep_row


def _post_body(pb_ref, pr_ref, aux_ref, aff_ref, out_ref):
    lane = lax.broadcasted_iota(jnp.int32, (1, 128), 1)

    def affc(k):
        return jnp.max(jnp.where(lane == k, aff_ref[0:1, :], 0.0), axis=1,
                       keepdims=True)

    a00, a01, a02 = affc(0), affc(1), affc(2)
    a10, a11, a12 = affc(3), affc(4), affc(5)

    def pre2samp_rows(ref):
        r = ref[:, :]
        ctx = r[0:1, :] * RWH
        cty = r[1:2, :] * RWH
        tx = (a00 * ctx + a01 * cty) + a02
        ty = (a10 * ctx + a11 * cty) + a12
        am0 = (r[2:3, :] * RWH) * a00
        am1 = (r[3:4, :] * RWH) * a11
        am2 = (r[4:5, :] * RWH) * a00
        am3 = (r[5:6, :] * RWH) * a11
        return tx, ty, am0, am1, am2, am3

    tx, ty, am0, am1, am2, am3 = pre2samp_rows(pb_ref)
    ptx, pty, pm0, pm1, pm2, pm3 = pre2samp_rows(pr_ref)

    scores = jax.nn.sigmoid(aux_ref[0:1, :])
    keep = aux_ref[1:2, :]

    rows = jnp.concatenate([
        scores,
        tx - am0, ty - am1, tx + am2, ty + am3,
        tx, ty,
        ptx - pm0, pty - pm1, ptx + pm2, pty + pm3,
        jnp.zeros((5, pb_ref.shape[1]), jnp.float32),
    ], axis=0) * keep
    out_ref[:, :] = rows


def kernel(pred_logits, pred_boxes, pre_boxes, src_valid_trans, patch_area):
    logit1 = pred_logits[0, :, 1]
    scores_nms = jax.nn.sigmoid(logit1)
    order = jnp.argsort(-scores_nms)

    pb0 = pred_boxes[0]
    nnpp = jnp.array([-1.0, -1.0, 1.0, 1.0], jnp.float32)
    rect_copy = pb0[:, 2:6] * nnpp + jnp.tile(pb0[:, 0:2], (1, 2))
    rect_sorted = rect_copy[order]
    rect_pad = jnp.zeros((NP, 4), jnp.float32).at[:N].set(rect_sorted)
    bs = rect_pad.T.reshape(4, NB, B).transpose(1, 0, 2)
    bs = jnp.concatenate([bs, jnp.zeros((NB, 4, B), jnp.float32)], axis=1)

    keep_blocks = pl.pallas_call(
        _nms_body,
        grid=(NB,),
        in_specs=[pl.BlockSpec((NB, 8, B), lambda i: (0, 0, 0))],
        out_specs=pl.BlockSpec((NB, B), lambda i: (0, 0)),
        out_shape=jax.ShapeDtypeStruct((NB, B), jnp.float32),
    )(bs)

    keep_sorted = keep_blocks.reshape(NP)[:N]
    keep = jnp.zeros((N,), jnp.float32).at[order].set(keep_sorted)

    def chan_major(x):
        t = jnp.zeros((8, NP), jnp.float32)
        return t.at[:6, :N].set(x.T)

    pbT = chan_major(pb0)
    prT = chan_major(pre_boxes[0])
    aux = jnp.zeros((8, NP), jnp.float32).at[0, :N].set(logit1).at[1, :N].set(keep)
    aff = jnp.zeros((8, 128), jnp.float32).at[0, :6].set(
        src_valid_trans[0, 0].reshape(6))

    res = pl.pallas_call(
        _post_body,
        in_specs=[pl.BlockSpec((8, NP), lambda: (0, 0)),
                  pl.BlockSpec((8, NP), lambda: (0, 0)),
                  pl.BlockSpec((8, NP), lambda: (0, 0)),
                  pl.BlockSpec((8, 128), lambda: (0, 0))],
        out_specs=pl.BlockSpec((16, NP), lambda: (0, 0)),
        out_shape=jax.ShapeDtypeStruct((16, NP), jnp.float32),
    )(pbT, prT, aux, aff)

    out = res[:11, :N].T[None]
    return out, patch_area
